# baseline (speedup 1.0000x reference)
import jax
import jax.numpy as jnp
from jax.experimental import pallas as pl
from jax.experimental.pallas import tpu as pltpu

_BLOCK_ROWS = 1024
_NBUF = 3


def _router_body(x_hbm, w1_ref, b1_ref, w2_ref, b2_ref, o_ref, xbuf, sems):
    n_tokens = x_hbm.shape[0]
    block = _BLOCK_ROWS
    nblk = n_tokens // block
    rows_per_blk = block // 128

    def issue(i, slot):
        pltpu.make_async_copy(
            x_hbm.at[pl.ds(i * block, block), :],
            xbuf.at[slot],
            sems.at[slot],
        ).start()

    def wait(i, slot):
        pltpu.make_async_copy(
            x_hbm.at[pl.ds(i * block, block), :],
            xbuf.at[slot],
            sems.at[slot],
        ).wait()

    for i in range(_NBUF):
        issue(i, i)

    for i in range(nblk):
        slot = i % _NBUF
        wait(i, slot)
        h = jnp.dot(xbuf[slot], w1_ref[...], preferred_element_type=jnp.float32)
        h = jnp.maximum(h + b1_ref[...], 0.0)
        logits = jnp.sum(h * w2_ref[...], axis=1, keepdims=True) + b2_ref[...]
        probs = jax.nn.sigmoid(logits)
        o_ref[pl.ds(i * rows_per_blk, rows_per_blk), :] = probs.reshape(
            rows_per_blk, 128
        )
        if i + _NBUF < nblk:
            issue(i + _NBUF, slot)


def kernel(x, W1, b1, W2, b2):
    n_tokens, input_dim = x.shape
    hidden_dim = W1.shape[1]

    w1b = W1.astype(jnp.bfloat16)
    b1r = b1.reshape(1, hidden_dim)
    w2r = W2.reshape(1, hidden_dim)
    b2r = b2.reshape(1, 1)

    out = pl.pallas_call(
        _router_body,
        in_specs=[
            pl.BlockSpec(memory_space=pl.ANY),
            pl.BlockSpec(memory_space=pltpu.VMEM),
            pl.BlockSpec(memory_space=pltpu.VMEM),
            pl.BlockSpec(memory_space=pltpu.VMEM),
            pl.BlockSpec(memory_space=pltpu.VMEM),
        ],
        out_specs=pl.BlockSpec(memory_space=pltpu.VMEM),
        out_shape=jax.ShapeDtypeStruct((n_tokens // 128, 128), jnp.float32),
        scratch_shapes=[
            pltpu.VMEM((_NBUF, _BLOCK_ROWS, input_dim), jnp.float32),
            pltpu.SemaphoreType.DMA((_NBUF,)),
        ],
    )(x, w1b, b1r, w2r, b2r)
    return out.reshape(n_tokens, 1)


# speedup vs baseline: 1.1479x; 1.1479x over previous
import jax
import jax.numpy as jnp
from jax.experimental import pallas as pl
from jax.experimental.pallas import tpu as pltpu

_BLOCK_ROWS = 1024


def _router_body(x_ref, w1_ref, b1_ref, w2_ref, b2_ref, o_ref):
    h = jnp.dot(x_ref[...], w1_ref[...], preferred_element_type=jnp.float32)
    h = jnp.maximum(h + b1_ref[...], 0.0)
    logits = jnp.sum(h * w2_ref[...], axis=1, keepdims=True) + b2_ref[...]
    probs = jax.nn.sigmoid(logits)
    o_ref[...] = probs.reshape(o_ref.shape)


def kernel(x, W1, b1, W2, b2):
    n_tokens, input_dim = x.shape
    hidden_dim = W1.shape[1]
    block = _BLOCK_ROWS
    grid = n_tokens // block
    orows = block // 128

    w1b = W1.astype(jnp.bfloat16)
    b1r = b1.reshape(1, hidden_dim)
    w2r = W2.reshape(1, hidden_dim)
    b2r = b2.reshape(1, 1)

    out = pl.pallas_call(
        _router_body,
        grid=(grid,),
        in_specs=[
            pl.BlockSpec((block, input_dim), lambda i: (i, 0)),
            pl.BlockSpec((input_dim, hidden_dim), lambda i: (0, 0)),
            pl.BlockSpec((1, hidden_dim), lambda i: (0, 0)),
            pl.BlockSpec((1, hidden_dim), lambda i: (0, 0)),
            pl.BlockSpec((1, 1), lambda i: (0, 0)),
        ],
        out_specs=pl.BlockSpec((orows, 128), lambda i: (i, 0)),
        out_shape=jax.ShapeDtypeStruct((n_tokens // 128, 128), jnp.float32),
        compiler_params=pltpu.CompilerParams(
            dimension_semantics=("parallel",),
        ),
    )(x, w1b, b1r, w2r, b2r)
    return out.reshape(n_tokens, 1)
